# baseline (speedup 1.0000x reference)
import functools

import jax
import jax.numpy as jnp
from jax import lax
from jax.experimental import pallas as pl
from jax.experimental.pallas import tpu as pltpu
from jax.experimental.pallas import tpu_sc as plsc

_VOCAB = 1000000
_HID = 64
_ROW32 = _HID // 2
_NUM_IDX = 4096 * 200
_NC, _NS = 2, 16
_NW = _NC * _NS
_PER_W = _NUM_IDX // _NW
_CHUNK = 128
_NCHUNK = _PER_W // _CHUNK


def _gather_body(idx_hbm, tab_hbm, out_hbm, idx_v, rows_v, sem):
    wid = lax.axis_index("s") * _NC + lax.axis_index("c")
    base = wid * _PER_W
    pltpu.sync_copy(idx_hbm.at[pl.ds(base, _PER_W)], idx_v)

    def step(c, carry):
        off = c * _CHUNK
        pltpu.async_copy(
            tab_hbm.at[idx_v.at[pl.ds(off, _CHUNK)]], rows_v, sem
        ).wait()
        pltpu.sync_copy(rows_v, out_hbm.at[pl.ds(base + off, _CHUNK)])
        return carry

    lax.fori_loop(0, _NCHUNK, step, 0)


_gather = pl.kernel(
    _gather_body,
    out_type=jax.ShapeDtypeStruct((_NUM_IDX, _ROW32), jnp.int32),
    mesh=plsc.VectorSubcoreMesh(core_axis_name="c", subcore_axis_name="s"),
    scratch_types=[
        pltpu.VMEM((_PER_W,), jnp.int32),
        pltpu.VMEM((_CHUNK, _ROW32), jnp.int32),
        pltpu.SemaphoreType.DMA,
    ],
    compiler_params=pltpu.CompilerParams(use_tc_tiling_on_sc=False),
)


_BLK = 4096


def _proj_body(h_ref, w_ref, b_ref, out_ref):
    acc = lax.dot_general(
        h_ref[...], w_ref[...], (((1,), (1,)), ((), ())),
        preferred_element_type=jnp.float32,
    )
    out_ref[...] = (acc + b_ref[...].astype(jnp.float32)).astype(jnp.bfloat16)


_proj = pl.pallas_call(
    _proj_body,
    grid=(_NUM_IDX // _BLK,),
    in_specs=[
        pl.BlockSpec((_BLK, _HID), lambda i: (i, 0)),
        pl.BlockSpec((_HID, _HID), lambda i: (0, 0)),
        pl.BlockSpec((1, _HID), lambda i: (0, 0)),
    ],
    out_specs=pl.BlockSpec((_BLK, _HID), lambda i: (i, 0)),
    out_shape=jax.ShapeDtypeStruct((_NUM_IDX, _HID), jnp.bfloat16),
)


def kernel(x, table, W, b):
    bsz, hist = x.shape
    idx = x.reshape(-1).astype(jnp.int32)
    tab32 = lax.bitcast_convert_type(
        table.reshape(_VOCAB, _ROW32, 2), jnp.int32
    )
    h32 = _gather(idx, tab32)
    h = lax.bitcast_convert_type(h32, jnp.bfloat16).reshape(_NUM_IDX, _HID)
    out = _proj(h, W, b.reshape(1, _HID))
    return out.reshape(bsz, hist, _HID)


# speedup vs baseline: 1.5221x; 1.5221x over previous
import functools

import jax
import jax.numpy as jnp
from jax import lax
from jax.experimental import pallas as pl
from jax.experimental.pallas import tpu as pltpu
from jax.experimental.pallas import tpu_sc as plsc

_VOCAB = 1000000
_HID = 64
_ROW32 = _HID // 2
_NUM_IDX = 4096 * 200
_NC, _NS = 2, 16
_NW = _NC * _NS
_PER_W = _NUM_IDX // _NW
_CHUNK = 128
_K = 10
_BURST = _K * _CHUNK
_NBIG = _PER_W // _BURST
_NBUF = 2


def _gather_body(idx_hbm, tab32, out_hbm, idx_v, rows_v, gsem, wsem):
    wid = lax.axis_index("s") * _NC + lax.axis_index("c")
    base = wid * _PER_W
    pltpu.sync_copy(idx_hbm.at[pl.ds(base, _PER_W)], idx_v)

    def wb_copy(b, row0):
        off = pl.multiple_of(base + row0, _BURST)
        return pltpu.make_async_copy(
            rows_v.at[b], out_hbm.at[pl.ds(off, _BURST)], wsem.at[b]
        )

    def gather_copy(b, row0, j):
        return pltpu.make_async_copy(
            tab32.at[idx_v.at[pl.ds(row0 + j * _CHUNK, _CHUNK)]],
            rows_v.at[b, pl.ds(j * _CHUNK, _CHUNK)],
            gsem,
        )

    def burst_pair(i, carry):
        for b in range(_NBUF):
            row0 = (i * _NBUF + b) * _BURST
            @pl.when(i > 0)
            def _():
                wb_copy(b, row0).wait()

            for j in range(_K):
                gather_copy(b, row0, j).start()
            for j in range(_K):
                gather_copy(b, row0, j).wait()
            wb_copy(b, row0).start()
        return carry

    lax.fori_loop(0, _NBIG // _NBUF, burst_pair, 0)
    last = (_NBIG - _NBUF) * _BURST
    for b in range(_NBUF):
        wb_copy(b, last + b * _BURST).wait()


_gather = pl.kernel(
    _gather_body,
    out_type=jax.ShapeDtypeStruct((_NUM_IDX, _ROW32), jnp.int32),
    mesh=plsc.VectorSubcoreMesh(core_axis_name="c", subcore_axis_name="s"),
    scratch_types=[
        pltpu.VMEM((_PER_W,), jnp.int32),
        pltpu.VMEM((_NBUF, _BURST, _ROW32), jnp.int32),
        pltpu.SemaphoreType.DMA,
        pltpu.SemaphoreType.DMA((_NBUF,)),
    ],
    compiler_params=pltpu.CompilerParams(use_tc_tiling_on_sc=False),
)


_BLK = 4096


def _proj_body(h_ref, we_ref, wo_ref, b_ref, out_ref):
    h32 = h_ref[...]
    even = lax.bitcast_convert_type(h32 << 16, jnp.float32)
    odd = lax.bitcast_convert_type(
        h32 & jnp.int32(-65536), jnp.float32
    )
    acc = lax.dot_general(
        even.astype(jnp.bfloat16), we_ref[...], (((1,), (0,)), ((), ())),
        preferred_element_type=jnp.float32,
    )
    acc = acc + lax.dot_general(
        odd.astype(jnp.bfloat16), wo_ref[...], (((1,), (0,)), ((), ())),
        preferred_element_type=jnp.float32,
    )
    out_ref[...] = (acc + b_ref[...].astype(jnp.float32)).astype(jnp.bfloat16)


_proj = pl.pallas_call(
    _proj_body,
    grid=(_NUM_IDX // _BLK,),
    in_specs=[
        pl.BlockSpec((_BLK, _ROW32), lambda i: (i, 0)),
        pl.BlockSpec((_ROW32, _HID), lambda i: (0, 0)),
        pl.BlockSpec((_ROW32, _HID), lambda i: (0, 0)),
        pl.BlockSpec((1, _HID), lambda i: (0, 0)),
    ],
    out_specs=pl.BlockSpec((_BLK, _HID), lambda i: (i, 0)),
    out_shape=jax.ShapeDtypeStruct((_NUM_IDX, _HID), jnp.bfloat16),
)


def kernel(x, table, W, b):
    bsz, hist = x.shape
    idx = x.reshape(-1).astype(jnp.int32)
    tab32 = lax.bitcast_convert_type(
        table.reshape(_VOCAB, _ROW32, 2), jnp.int32
    )
    h32 = _gather(idx, tab32)
    we = W[:, 0::2].T
    wo = W[:, 1::2].T
    out = _proj(h32, we, wo, b.reshape(1, _HID))
    return out.reshape(bsz, hist, _HID)


# speedup vs baseline: 2.2315x; 1.4661x over previous
import functools

import jax
import jax.numpy as jnp
from jax import lax
from jax.experimental import pallas as pl
from jax.experimental.pallas import tpu as pltpu
from jax.experimental.pallas import tpu_sc as plsc

_VOCAB = 1000000
_HID = 64
_ROW32 = _HID // 2
_NUM_IDX = 4096 * 200
_NC, _NS = 2, 16
_NW = _NC * _NS
_PER_W = _NUM_IDX // _NW
_CHUNK = 128
_K = 10
_BURST = _K * _CHUNK
_NBIG = _PER_W // _BURST
_NBUF = 2


def _gather_body(idx_hbm, tab_hbm, out_hbm, idx_v, rows_v, gsem, wsem):
    wid = lax.axis_index("s") * _NC + lax.axis_index("c")
    base = wid * _PER_W
    pltpu.sync_copy(idx_hbm.at[pl.ds(base, _PER_W)], idx_v)

    def wb_copy(b, row0):
        off = pl.multiple_of(base + row0, _BURST)
        return pltpu.make_async_copy(
            rows_v.at[b], out_hbm.at[pl.ds(off, _BURST)], wsem.at[b]
        )

    def gather_copy(b, row0, j):
        return pltpu.make_async_copy(
            tab_hbm.at[idx_v.at[pl.ds(row0 + j * _CHUNK, _CHUNK)]],
            rows_v.at[b, pl.ds(j * _CHUNK, _CHUNK)],
            gsem,
        )

    def burst_pair(i, carry):
        for b in range(_NBUF):
            row0 = (i * _NBUF + b) * _BURST
            @pl.when(i > 0)
            def _():
                wb_copy(b, row0).wait()

            for j in range(_K):
                gather_copy(b, row0, j).start()
            for j in range(_K):
                gather_copy(b, row0, j).wait()
            wb_copy(b, row0).start()
        return carry

    lax.fori_loop(0, _NBIG // _NBUF, burst_pair, 0)
    last = (_NBIG - _NBUF) * _BURST
    for b in range(_NBUF):
        wb_copy(b, last + b * _BURST).wait()


_gather = pl.kernel(
    _gather_body,
    out_type=jax.ShapeDtypeStruct((_NUM_IDX, _HID), jnp.bfloat16),
    mesh=plsc.VectorSubcoreMesh(core_axis_name="c", subcore_axis_name="s"),
    scratch_types=[
        pltpu.VMEM((_PER_W,), jnp.int32),
        pltpu.VMEM((_NBUF, _BURST, _HID), jnp.bfloat16),
        pltpu.SemaphoreType.DMA,
        pltpu.SemaphoreType.DMA((_NBUF,)),
    ],
    compiler_params=pltpu.CompilerParams(use_tc_tiling_on_sc=False),
)


_BLK = 4096


def _proj_body(h_ref, w_ref, b_ref, out_ref):
    acc = lax.dot_general(
        h_ref[...], w_ref[...], (((1,), (1,)), ((), ())),
        preferred_element_type=jnp.float32,
    )
    out_ref[...] = (acc + b_ref[...].astype(jnp.float32)).astype(jnp.bfloat16)


_proj = pl.pallas_call(
    _proj_body,
    grid=(_NUM_IDX // _BLK,),
    in_specs=[
        pl.BlockSpec((_BLK, _HID), lambda i: (i, 0)),
        pl.BlockSpec((_HID, _HID), lambda i: (0, 0)),
        pl.BlockSpec((1, _HID), lambda i: (0, 0)),
    ],
    out_specs=pl.BlockSpec((_BLK, _HID), lambda i: (i, 0)),
    out_shape=jax.ShapeDtypeStruct((_NUM_IDX, _HID), jnp.bfloat16),
)


def kernel(x, table, W, b):
    bsz, hist = x.shape
    idx = x.reshape(-1).astype(jnp.int32)
    h = _gather(idx, table)
    out = _proj(h, W, b.reshape(1, _HID))
    return out.reshape(bsz, hist, _HID)
